# baseline (speedup 1.0000x reference)
import functools

import jax
import jax.numpy as jnp
from jax import lax
from jax.experimental import pallas as pl
from jax.experimental.pallas import tpu as pltpu
from jax.experimental.pallas import tpu_sc as plsc

N_NODES = 10000
N_EDGES = 160000
NUM_GRAPHS = 128

NTILES = 32
EPAD = 163840
EPT = EPAD // NTILES
K2 = 128
NPAD = 10016
NHALF = 5008
ACCR = 5024
NZB = 157

_f32 = jnp.float32
_i32 = jnp.int32

_MESH = plsc.VectorSubcoreMesh(core_axis_name="c", subcore_axis_name="s")


KQ = 2


def _gather_body(nchunks, refs):
    tabs = refs[:nchunks]
    idx2_hbm = refs[nchunks]
    out = refs[nchunks + 1]
    idx2 = refs[nchunks + 2]
    rbufs = refs[nchunks + 3:nchunks + 3 + KQ]
    gsem, wsem = refs[nchunks + 3 + KQ], refs[nchunks + 4 + KQ]
    cid = lax.axis_index("c")
    sid = lax.axis_index("s")
    wid = cid * 16 + sid
    nb = EPT // K2

    pltpu.sync_copy(idx2_hbm.at[pl.ds(wid * nb, nb)], idx2)
    for cc in range(nchunks):
        def sb(q, _):
            ds = [pltpu.async_copy(tabs[cc].at[idx2.at[q * KQ + j]],
                                   rbufs[j], gsem) for j in range(KQ)]
            for d in ds:
                d.wait()
            ws = [pltpu.async_copy(
                rbufs[j],
                out.at[cc, pl.ds(wid * EPT + (q * KQ + j) * K2, K2)],
                wsem) for j in range(KQ)]
            for w in ws:
                w.wait()
            return 0

        lax.fori_loop(0, nb // KQ, sb, 0)


def _gather_call(tabs, idx2, dc):
    nchunks = len(tabs)

    def wrapped(*refs):
        _gather_body(nchunks, refs)

    return pl.kernel(
        wrapped,
        out_type=jax.ShapeDtypeStruct((nchunks, EPAD, dc), _f32),
        mesh=_MESH,
        scratch_types=[
            pltpu.VMEM((EPAD // K2 // NTILES, K2), _i32),
        ] + [pltpu.VMEM((K2, dc), _f32) for _ in range(KQ)] + [
            pltpu.SemaphoreType.DMA,
            pltpu.SemaphoreType.DMA,
        ],
    )(*tabs, idx2)


def _gs_body(nchunks, refs):
    tabs = refs[:nchunks]
    src2_hbm = refs[nchunks]
    idxl_hbm = refs[nchunks + 1]
    idxh_hbm = refs[nchunks + 2]
    out = refs[nchunks + 3]
    src2 = refs[nchunks + 4]
    idxa, idxb = refs[nchunks + 5], refs[nchunks + 6]
    rbufs = refs[nchunks + 7:nchunks + 7 + KQ]
    zblk = refs[nchunks + 7 + KQ]
    acc_s = refs[nchunks + 8 + KQ]
    gsem, ssem = refs[nchunks + 9 + KQ], refs[nchunks + 10 + KQ]
    idx_v = (idxa, idxb)
    cid = lax.axis_index("c")
    sid = lax.axis_index("s")
    wid = cid * 16 + sid
    nb = EPT // K2

    pltpu.sync_copy(src2_hbm.at[pl.ds(wid * nb, nb)], src2)
    pltpu.sync_copy(idxl_hbm.at[pl.ds(wid * nb, nb)], idxa)
    pltpu.sync_copy(idxh_hbm.at[pl.ds(wid * nb, nb)], idxb)

    def zb(i, _):
        for q in range(DC // 16):
            zblk[i, pl.ds(q * 16, 16)] = jnp.zeros((16,), _f32)
        return 0

    lax.fori_loop(0, NZB, zb, 0)

    for cc in range(nchunks):
        for hf in range(2):
            for r2 in range(2):
                pltpu.sync_copy(
                    zblk, acc_s.at[pl.ds((sid * 2 + r2) * NZB, NZB)])
            plsc.subcore_barrier()

            def sb(q, _):
                ds = [pltpu.async_copy(tabs[cc].at[src2.at[q * KQ + j]],
                                       rbufs[j], gsem) for j in range(KQ)]
                for d in ds:
                    d.wait()
                ws = [pltpu.async_copy(
                    rbufs[j], acc_s.at[idx_v[hf].at[q * KQ + j]],
                    ssem, add=True) for j in range(KQ)]
                for w in ws:
                    w.wait()
                return 0

            lax.fori_loop(0, nb // KQ, sb, 0)
            plsc.subcore_barrier()

            @pl.when(sid == 0)
            def _():
                pltpu.sync_copy(acc_s, out.at[cc, cid, hf])

            plsc.subcore_barrier()


def _gs_call(tabs, src2, idx_lo, idx_hi):
    nchunks = len(tabs)

    def wrapped(*refs):
        _gs_body(nchunks, refs)

    return pl.kernel(
        wrapped,
        out_type=jax.ShapeDtypeStruct((nchunks, 2, 2, ACCR, DC), _f32),
        mesh=_MESH,
        scratch_types=[
            pltpu.VMEM((EPAD // K2 // NTILES, K2), _i32),
            pltpu.VMEM((EPAD // K2 // NTILES, K2), _i32),
            pltpu.VMEM((EPAD // K2 // NTILES, K2), _i32),
        ] + [pltpu.VMEM((K2, DC), _f32) for _ in range(KQ)] + [
            pltpu.VMEM((NZB, DC), _f32),
            pltpu.VMEM_SHARED((ACCR, DC), _f32),
            pltpu.SemaphoreType.DMA,
            pltpu.SemaphoreType.DMA,
        ],
    )(*tabs, src2, idx_lo, idx_hi)


def _scatter_body(nchunks, refs):
    wg_hbm = refs[0]
    idx_hbm = (refs[1], refs[2])
    out = refs[3]
    idxa, idxb = refs[4], refs[5]
    rbufs = refs[6:6 + KQ]
    zblk, acc_s = refs[6 + KQ], refs[7 + KQ]
    gsem, ssem = refs[8 + KQ], refs[9 + KQ]
    idx_v = (idxa, idxb)
    cid = lax.axis_index("c")
    sid = lax.axis_index("s")
    wid = cid * 16 + sid
    nb = EPT // K2

    for hf in range(2):
        pltpu.sync_copy(idx_hbm[hf].at[pl.ds(wid * nb, nb)], idx_v[hf])

    def zb(i, _):
        for q in range(DC // 16):
            zblk[i, pl.ds(q * 16, 16)] = jnp.zeros((16,), _f32)
        return 0

    lax.fori_loop(0, NZB, zb, 0)

    for cc in range(nchunks):
        for hf in range(2):
            for r2 in range(2):
                pltpu.sync_copy(
                    zblk, acc_s.at[pl.ds((sid * 2 + r2) * NZB, NZB)])
            plsc.subcore_barrier()

            def sb(q, _):
                ds = [pltpu.async_copy(
                    wg_hbm.at[cc, pl.ds(wid * EPT + (q * KQ + j) * K2, K2)],
                    rbufs[j], gsem) for j in range(KQ)]
                for d in ds:
                    d.wait()
                ws = [pltpu.async_copy(
                    rbufs[j], acc_s.at[idx_v[hf].at[q * KQ + j]],
                    ssem, add=True) for j in range(KQ)]
                for w in ws:
                    w.wait()
                return 0

            lax.fori_loop(0, nb // KQ, sb, 0)
            plsc.subcore_barrier()

            @pl.when(sid == 0)
            def _():
                pltpu.sync_copy(acc_s, out.at[cc, cid, hf])

            plsc.subcore_barrier()


def _scatter_call(wg, idx_lo, idx_hi):
    nchunks = wg.shape[0]

    def wrapped(*refs):
        _scatter_body(nchunks, refs)

    return pl.kernel(
        wrapped,
        out_type=jax.ShapeDtypeStruct((nchunks, 2, 2, ACCR, DC), _f32),
        mesh=_MESH,
        scratch_types=[
            pltpu.VMEM((EPAD // K2 // NTILES, K2), _i32),
            pltpu.VMEM((EPAD // K2 // NTILES, K2), _i32),
        ] + [pltpu.VMEM((K2, DC), _f32) for _ in range(KQ)] + [
            pltpu.VMEM((NZB, DC), _f32),
            pltpu.VMEM_SHARED((ACCR, DC), _f32),
            pltpu.SemaphoreType.DMA,
            pltpu.SemaphoreType.DMA,
        ],
    )(wg, idx_lo, idx_hi)


def _half_idx(dst_flat):
    lo = jnp.where(dst_flat < NHALF, dst_flat, NHALF)
    inhi = (dst_flat >= NHALF) & (dst_flat < N_NODES)
    hi = jnp.where(inhi, dst_flat - NHALF, NHALF)
    shp = (EPAD // K2, K2)
    return lo.reshape(shp), hi.reshape(shp)


def _merge_halves(acc):
    acc = acc.sum(axis=1)
    return jnp.concatenate([acc[:, 0, :NHALF], acc[:, 1, :N_NODES - NHALF]], axis=1)


DC = 128


def _chunk(a):
    do = a.shape[1]
    return [a[:, i * DC:(i + 1) * DC] for i in range(do // DC)]


def _aug(a):
    dop = max(DC, a.shape[1])
    return jnp.concatenate([
        jnp.pad(a, ((0, 0), (0, dop - a.shape[1]))),
        jnp.zeros((NPAD - N_NODES, dop), _f32)])


def _merge_num(num_p):
    num = num_p.sum(axis=1)[:, :N_NODES]
    return jnp.moveaxis(num, 0, 1).reshape(N_NODES, -1)


def _leaky(z):
    return 0.6 * z + 0.4 * jnp.abs(z)


def _gat_layer(h, src2, dst2, p, do):
    dop = max(DC, do)
    attp = jnp.pad(p['att'], (0, dop - do))
    xl = h @ p['Wl'] + p['bl']
    xr = h @ p['Wr'] + p['br']
    xla = _aug(xl)
    xra = _aug(xr)
    xlg = _gather_call(_chunk(xla), src2, DC)
    xrg = _gather_call(_chunk(xra), dst2, DC)
    xlg_f = jnp.moveaxis(xlg, 0, 1).reshape(EPAD, dop)
    zg = xlg_f + jnp.moveaxis(xrg, 0, 1).reshape(EPAD, dop)
    ex = jnp.exp(_leaky(zg) @ attp)
    wg = ex[:, None] * xlg_f
    wg_c = jnp.moveaxis(wg.reshape(EPAD, dop // DC, DC), 1, 0)
    den_c = jnp.broadcast_to(ex[:, None], (EPAD, DC))[None]
    acc = _scatter_call(jnp.concatenate([wg_c, den_c]), *_half_idx(_DSTF[0]))
    mh = _merge_halves(acc)
    num = jnp.moveaxis(mh[:-1], 0, 1).reshape(N_NODES, -1)[:, :do]
    den = mh[-1][:, 0]
    exs = jnp.exp(_leaky(xl + xr) @ p['att'])
    out = (num + exs[:, None] * xl) / (den + exs + 1e-16)[:, None]
    return out + p['bias']


def _arma_layer(g, x_skip, dis, src2, dst2, p, do):
    hp = _aug(dis[:, None] * (g @ p['W']))
    hg = _gather_call(_chunk(hp), src2, DC)
    acc = _scatter_call(hg, *_half_idx(_DSTF[0]))
    mh = _merge_halves(acc)
    agg = dis[:, None] * jnp.moveaxis(mh, 0, 1).reshape(N_NODES, -1)[:, :do]
    return jax.nn.relu(agg + x_skip @ p['V'] + p['bias'])


def _head_body(pool_ref, sa_ref, af_ref, w1, b1, w2, b2, w4, b4, w3, out_ref):
    z1 = jnp.dot(pool_ref[...], w1[...], preferred_element_type=_f32) + b1[...]
    z2 = jnp.dot(sa_ref[...], w2[...], preferred_element_type=_f32) + b2[...]
    z3 = jnp.dot(af_ref[...], w4[...], preferred_element_type=_f32) + b4[...]
    w3a = w3[...][:512]
    w3b = w3[...][512:1024]
    w3c = w3[...][1024:]
    out_ref[...] = (
        jnp.dot(z1, w3a, preferred_element_type=_f32)
        + jnp.dot(z2, w3b, preferred_element_type=_f32)
        + jnp.dot(z3, w3c, preferred_element_type=_f32)
    )


def _head(pool, sa, af, p):
    return pl.pallas_call(
        _head_body,
        out_shape=jax.ShapeDtypeStruct((NUM_GRAPHS, 1), _f32),
    )(pool, sa, af,
      p['lin1']['W'], p['lin1']['b'],
      p['lin2']['W'], p['lin2']['b'],
      p['lin4']['W'], p['lin4']['b'],
      p['lin3']['W'])


def _seg_sum(d, s, n):
    return jax.ops.segment_sum(d, s, num_segments=n)


def _seg_mean(d, s, n):
    tot = jax.ops.segment_sum(d, s, num_segments=n)
    cnt = jax.ops.segment_sum(jnp.ones(d.shape[:1], d.dtype), s, num_segments=n)
    cnt = jnp.maximum(cnt, 1.0)
    return tot / cnt.reshape((-1,) + (1,) * (d.ndim - 1))


def _seg_max(d, s, n):
    m = jax.ops.segment_max(d, s, num_segments=n)
    return jnp.where(jnp.isfinite(m), m, 0.0)


def _hmm(a, b):
    return jnp.matmul(a, b, precision=jax.lax.Precision.HIGHEST)


def _gnorm(x, batch_oh, inv_cnt, p, b):
    mean = _hmm(batch_oh, x) * inv_cnt[:, None]
    out = x - _hmm(batch_oh.T, mean) * p['mean_scale']
    var = _hmm(batch_oh, out * out) * inv_cnt[:, None]
    scale = jax.lax.rsqrt(var + 1e-5) * p['weight']
    return out * _hmm(batch_oh.T, scale) + p['bias']


def _sort_aggr(x, batch, b, k):
    order = jnp.lexsort((-x[:, -1], batch))
    xs = x[order]
    bs = batch[order]
    counts = jnp.bincount(batch, length=b)
    starts = jnp.concatenate([jnp.zeros((1,), counts.dtype), jnp.cumsum(counts)[:-1]])
    rank = jnp.arange(x.shape[0]) - starts[bs]
    mask = (rank < k)[:, None]
    vals = jnp.where(mask, xs, 0.0)
    out = jnp.zeros((b, k, x.shape[1]), x.dtype).at[bs, jnp.clip(rank, 0, k - 1)].add(vals)
    return out.reshape(b, k * x.shape[1])


_DSTF = [None]


def kernel(x, edge_index, batch, additional_feat, params):
    b = NUM_GRAPHS
    src, dst = edge_index[0], edge_index[1]
    pad = jnp.full((EPAD - N_EDGES,), N_NODES, _i32)
    src2 = jnp.concatenate([src, pad]).reshape(EPAD // K2, K2)
    dst_flat = jnp.concatenate([dst, pad])
    dst2 = dst_flat.reshape(EPAD // K2, K2)
    _DSTF[0] = dst_flat
    batch_oh = (batch[None, :] == jnp.arange(b, dtype=batch.dtype)[:, None]
                ).astype(_f32)
    cnt = batch_oh.sum(axis=1)
    inv_cnt = 1.0 / jnp.maximum(cnt, 1.0)

    ones_rows = jnp.ones((1, EPAD, DC), _f32)
    deg = _merge_halves(_scatter_call(ones_rows, *_half_idx(dst_flat)))[0, :, 0]
    dis = jnp.where(deg > 0, 1.0 / jnp.sqrt(jnp.maximum(deg, 1e-12)), 0.0)

    h = _gnorm(jax.nn.elu(_gat_layer(x, src2, dst2, params['gat1'], 64)),
               batch_oh, inv_cnt, params['gn1'], b)
    h = _gnorm(jax.nn.elu(_gat_layer(h, src2, dst2, params['gat2'], 128)),
               batch_oh, inv_cnt, params['gn2'], b)
    h = _gnorm(jax.nn.elu(_gat_layer(h, src2, dst2, params['gat3'], 512)),
               batch_oh, inv_cnt, params['gn3'], b)
    g = _gnorm(jax.nn.elu(_arma_layer(x, x, dis, src2, dst2, params['arma1'], 64)),
               batch_oh, inv_cnt, params['gn4'], b)
    g = _gnorm(jax.nn.elu(_arma_layer(g, g, dis, src2, dst2, params['arma2'], 128)),
               batch_oh, inv_cnt, params['gn5'], b)
    g = _gnorm(jax.nn.elu(_arma_layer(g, g, dis, src2, dst2, params['arma3'], 512)),
               batch_oh, inv_cnt, params['gn6'], b)
    gg = jnp.concatenate([h, g], axis=1)
    sums = _hmm(batch_oh, gg)
    pool = jnp.concatenate([_seg_max(gg, batch, b), sums * inv_cnt[:, None],
                            sums], axis=1)
    sa = _sort_aggr(gg, batch, b, 4)
    return _head(pool, sa, additional_feat.reshape(b, 9), params)


# speedup vs baseline: 1.0042x; 1.0042x over previous
import functools

import jax
import jax.numpy as jnp
from jax import lax
from jax.experimental import pallas as pl
from jax.experimental.pallas import tpu as pltpu
from jax.experimental.pallas import tpu_sc as plsc

N_NODES = 10000
N_EDGES = 160000
NUM_GRAPHS = 128

NTILES = 32
EPAD = 163840
EPT = EPAD // NTILES
K2 = 128
NPAD = 10016
NHALF = 5008
ACCR = 5024
NZB = 157

_f32 = jnp.float32
_i32 = jnp.int32

_MESH = plsc.VectorSubcoreMesh(core_axis_name="c", subcore_axis_name="s")


KQ = 2


def _gather_body(nchunks, nidx, which, refs):
    tabs = refs[:nchunks]
    idx_hbm = refs[nchunks:nchunks + nidx]
    out = refs[nchunks + nidx]
    idx_v = refs[nchunks + nidx + 1:nchunks + nidx + 1 + nidx]
    o = nchunks + nidx + 1 + nidx
    rbufs = refs[o:o + KQ]
    gsem, wsem = refs[o + KQ], refs[o + KQ + 1]
    cid = lax.axis_index("c")
    sid = lax.axis_index("s")
    wid = cid * 16 + sid
    nb = EPT // K2

    for i in range(nidx):
        pltpu.sync_copy(idx_hbm[i].at[pl.ds(wid * nb, nb)], idx_v[i])
    for cc in range(nchunks):
        idx2 = idx_v[which[cc]]

        def sb(q, _):
            ds = [pltpu.async_copy(tabs[cc].at[idx2.at[q * KQ + j]],
                                   rbufs[j], gsem) for j in range(KQ)]
            for d in ds:
                d.wait()
            ws = [pltpu.async_copy(
                rbufs[j],
                out.at[cc, pl.ds(wid * EPT + (q * KQ + j) * K2, K2)],
                wsem) for j in range(KQ)]
            for w in ws:
                w.wait()
            return 0

        lax.fori_loop(0, nb // KQ, sb, 0)


def _gather_call(tabs, idx2s, which, dc):
    nchunks = len(tabs)
    nidx = len(idx2s)

    def wrapped(*refs):
        _gather_body(nchunks, nidx, which, refs)

    return pl.kernel(
        wrapped,
        out_type=jax.ShapeDtypeStruct((nchunks, EPAD, dc), _f32),
        mesh=_MESH,
        scratch_types=[
            pltpu.VMEM((EPAD // K2 // NTILES, K2), _i32)
            for _ in range(nidx)
        ] + [pltpu.VMEM((K2, dc), _f32) for _ in range(KQ)] + [
            pltpu.SemaphoreType.DMA,
            pltpu.SemaphoreType.DMA,
        ],
    )(*tabs, *idx2s)


def _gs_body(nchunks, refs):
    tabs = refs[:nchunks]
    src2_hbm = refs[nchunks]
    idxl_hbm = refs[nchunks + 1]
    idxh_hbm = refs[nchunks + 2]
    out = refs[nchunks + 3]
    src2 = refs[nchunks + 4]
    idxa, idxb = refs[nchunks + 5], refs[nchunks + 6]
    rbufs = refs[nchunks + 7:nchunks + 7 + KQ]
    zblk = refs[nchunks + 7 + KQ]
    acc_s = refs[nchunks + 8 + KQ]
    gsem, ssem = refs[nchunks + 9 + KQ], refs[nchunks + 10 + KQ]
    idx_v = (idxa, idxb)
    cid = lax.axis_index("c")
    sid = lax.axis_index("s")
    wid = cid * 16 + sid
    nb = EPT // K2

    pltpu.sync_copy(src2_hbm.at[pl.ds(wid * nb, nb)], src2)
    pltpu.sync_copy(idxl_hbm.at[pl.ds(wid * nb, nb)], idxa)
    pltpu.sync_copy(idxh_hbm.at[pl.ds(wid * nb, nb)], idxb)

    def zb(i, _):
        for q in range(DC // 16):
            zblk[i, pl.ds(q * 16, 16)] = jnp.zeros((16,), _f32)
        return 0

    lax.fori_loop(0, NZB, zb, 0)

    for cc in range(nchunks):
        for hf in range(2):
            for r2 in range(2):
                pltpu.sync_copy(
                    zblk, acc_s.at[pl.ds((sid * 2 + r2) * NZB, NZB)])
            plsc.subcore_barrier()

            def sb(q, _):
                ds = [pltpu.async_copy(tabs[cc].at[src2.at[q * KQ + j]],
                                       rbufs[j], gsem) for j in range(KQ)]
                for d in ds:
                    d.wait()
                ws = [pltpu.async_copy(
                    rbufs[j], acc_s.at[idx_v[hf].at[q * KQ + j]],
                    ssem, add=True) for j in range(KQ)]
                for w in ws:
                    w.wait()
                return 0

            lax.fori_loop(0, nb // KQ, sb, 0)
            plsc.subcore_barrier()

            @pl.when(sid == 0)
            def _():
                pltpu.sync_copy(acc_s, out.at[cc, cid, hf])

            plsc.subcore_barrier()


def _gs_call(tabs, src2, idx_lo, idx_hi):
    nchunks = len(tabs)

    def wrapped(*refs):
        _gs_body(nchunks, refs)

    return pl.kernel(
        wrapped,
        out_type=jax.ShapeDtypeStruct((nchunks, 2, 2, ACCR, DC), _f32),
        mesh=_MESH,
        scratch_types=[
            pltpu.VMEM((EPAD // K2 // NTILES, K2), _i32),
            pltpu.VMEM((EPAD // K2 // NTILES, K2), _i32),
            pltpu.VMEM((EPAD // K2 // NTILES, K2), _i32),
        ] + [pltpu.VMEM((K2, DC), _f32) for _ in range(KQ)] + [
            pltpu.VMEM((NZB, DC), _f32),
            pltpu.VMEM_SHARED((ACCR, DC), _f32),
            pltpu.SemaphoreType.DMA,
            pltpu.SemaphoreType.DMA,
        ],
    )(*tabs, src2, idx_lo, idx_hi)


def _scatter_body(nchunks, refs):
    wg_hbm = refs[0]
    idx_hbm = (refs[1], refs[2])
    out = refs[3]
    idxa, idxb = refs[4], refs[5]
    rbufs = refs[6:6 + KQ]
    zblk, acc_s = refs[6 + KQ], refs[7 + KQ]
    gsem, ssem = refs[8 + KQ], refs[9 + KQ]
    idx_v = (idxa, idxb)
    cid = lax.axis_index("c")
    sid = lax.axis_index("s")
    wid = cid * 16 + sid
    nb = EPT // K2

    for hf in range(2):
        pltpu.sync_copy(idx_hbm[hf].at[pl.ds(wid * nb, nb)], idx_v[hf])

    def zb(i, _):
        for q in range(DC // 16):
            zblk[i, pl.ds(q * 16, 16)] = jnp.zeros((16,), _f32)
        return 0

    lax.fori_loop(0, NZB, zb, 0)

    for cc in range(nchunks):
        for hf in range(2):
            for r2 in range(2):
                pltpu.sync_copy(
                    zblk, acc_s.at[pl.ds((sid * 2 + r2) * NZB, NZB)])
            plsc.subcore_barrier()

            def sb(q, _):
                ds = [pltpu.async_copy(
                    wg_hbm.at[cc, pl.ds(wid * EPT + (q * KQ + j) * K2, K2)],
                    rbufs[j], gsem) for j in range(KQ)]
                for d in ds:
                    d.wait()
                ws = [pltpu.async_copy(
                    rbufs[j], acc_s.at[idx_v[hf].at[q * KQ + j]],
                    ssem, add=True) for j in range(KQ)]
                for w in ws:
                    w.wait()
                return 0

            lax.fori_loop(0, nb // KQ, sb, 0)
            plsc.subcore_barrier()

            @pl.when(sid == 0)
            def _():
                pltpu.sync_copy(acc_s, out.at[cc, cid, hf])

            plsc.subcore_barrier()


def _scatter_call(wg, idx_lo, idx_hi):
    nchunks = wg.shape[0]

    def wrapped(*refs):
        _scatter_body(nchunks, refs)

    return pl.kernel(
        wrapped,
        out_type=jax.ShapeDtypeStruct((nchunks, 2, 2, ACCR, DC), _f32),
        mesh=_MESH,
        scratch_types=[
            pltpu.VMEM((EPAD // K2 // NTILES, K2), _i32),
            pltpu.VMEM((EPAD // K2 // NTILES, K2), _i32),
        ] + [pltpu.VMEM((K2, DC), _f32) for _ in range(KQ)] + [
            pltpu.VMEM((NZB, DC), _f32),
            pltpu.VMEM_SHARED((ACCR, DC), _f32),
            pltpu.SemaphoreType.DMA,
            pltpu.SemaphoreType.DMA,
        ],
    )(wg, idx_lo, idx_hi)


def _half_idx(dst_flat):
    lo = jnp.where(dst_flat < NHALF, dst_flat, NHALF)
    inhi = (dst_flat >= NHALF) & (dst_flat < N_NODES)
    hi = jnp.where(inhi, dst_flat - NHALF, NHALF)
    shp = (EPAD // K2, K2)
    return lo.reshape(shp), hi.reshape(shp)


def _merge_halves(acc):
    acc = acc.sum(axis=1)
    return jnp.concatenate([acc[:, 0, :NHALF], acc[:, 1, :N_NODES - NHALF]], axis=1)


DC = 128


def _chunk(a):
    do = a.shape[1]
    return [a[:, i * DC:(i + 1) * DC] for i in range(do // DC)]


def _aug(a):
    dop = max(DC, a.shape[1])
    return jnp.concatenate([
        jnp.pad(a, ((0, 0), (0, dop - a.shape[1]))),
        jnp.zeros((NPAD - N_NODES, dop), _f32)])


def _merge_num(num_p):
    num = num_p.sum(axis=1)[:, :N_NODES]
    return jnp.moveaxis(num, 0, 1).reshape(N_NODES, -1)


def _leaky(z):
    return 0.6 * z + 0.4 * jnp.abs(z)


def _gat_layer(h, src2, dst2, p, do):
    dop = max(DC, do)
    attp = jnp.pad(p['att'], (0, dop - do))
    xl = h @ p['Wl'] + p['bl']
    xr = h @ p['Wr'] + p['br']
    xla = _aug(xl)
    xra = _aug(xr)
    cl = _chunk(xla)
    cr = _chunk(xra)
    both = _gather_call(cl + cr, [src2, dst2],
                        [0] * len(cl) + [1] * len(cr), DC)
    xlg, xrg = both[:len(cl)], both[len(cl):]
    xlg_f = jnp.moveaxis(xlg, 0, 1).reshape(EPAD, dop)
    zg = xlg_f + jnp.moveaxis(xrg, 0, 1).reshape(EPAD, dop)
    ex = jnp.exp(_leaky(zg) @ attp)
    wg = ex[:, None] * xlg_f
    wg_c = jnp.moveaxis(wg.reshape(EPAD, dop // DC, DC), 1, 0)
    den_c = jnp.broadcast_to(ex[:, None], (EPAD, DC))[None]
    acc = _scatter_call(jnp.concatenate([wg_c, den_c]), *_half_idx(_DSTF[0]))
    mh = _merge_halves(acc)
    num = jnp.moveaxis(mh[:-1], 0, 1).reshape(N_NODES, -1)[:, :do]
    den = mh[-1][:, 0]
    exs = jnp.exp(_leaky(xl + xr) @ p['att'])
    out = (num + exs[:, None] * xl) / (den + exs + 1e-16)[:, None]
    return out + p['bias']


def _arma_layer(g, x_skip, dis, src2, dst2, p, do):
    hp = _aug(dis[:, None] * (g @ p['W']))
    hg = _gather_call(_chunk(hp), [src2], [0] * len(_chunk(hp)), DC)
    acc = _scatter_call(hg, *_half_idx(_DSTF[0]))
    mh = _merge_halves(acc)
    agg = dis[:, None] * jnp.moveaxis(mh, 0, 1).reshape(N_NODES, -1)[:, :do]
    return jax.nn.relu(agg + x_skip @ p['V'] + p['bias'])


def _head_body(pool_ref, sa_ref, af_ref, w1, b1, w2, b2, w4, b4, w3, out_ref):
    z1 = jnp.dot(pool_ref[...], w1[...], preferred_element_type=_f32) + b1[...]
    z2 = jnp.dot(sa_ref[...], w2[...], preferred_element_type=_f32) + b2[...]
    z3 = jnp.dot(af_ref[...], w4[...], preferred_element_type=_f32) + b4[...]
    w3a = w3[...][:512]
    w3b = w3[...][512:1024]
    w3c = w3[...][1024:]
    out_ref[...] = (
        jnp.dot(z1, w3a, preferred_element_type=_f32)
        + jnp.dot(z2, w3b, preferred_element_type=_f32)
        + jnp.dot(z3, w3c, preferred_element_type=_f32)
    )


def _head(pool, sa, af, p):
    return pl.pallas_call(
        _head_body,
        out_shape=jax.ShapeDtypeStruct((NUM_GRAPHS, 1), _f32),
    )(pool, sa, af,
      p['lin1']['W'], p['lin1']['b'],
      p['lin2']['W'], p['lin2']['b'],
      p['lin4']['W'], p['lin4']['b'],
      p['lin3']['W'])


def _seg_sum(d, s, n):
    return jax.ops.segment_sum(d, s, num_segments=n)


def _seg_mean(d, s, n):
    tot = jax.ops.segment_sum(d, s, num_segments=n)
    cnt = jax.ops.segment_sum(jnp.ones(d.shape[:1], d.dtype), s, num_segments=n)
    cnt = jnp.maximum(cnt, 1.0)
    return tot / cnt.reshape((-1,) + (1,) * (d.ndim - 1))


def _seg_max(d, s, n):
    m = jax.ops.segment_max(d, s, num_segments=n)
    return jnp.where(jnp.isfinite(m), m, 0.0)


def _hmm(a, b):
    return jnp.matmul(a, b, precision=jax.lax.Precision.HIGHEST)


def _gnorm(x, batch_oh, inv_cnt, p, b):
    mean = _hmm(batch_oh, x) * inv_cnt[:, None]
    out = x - _hmm(batch_oh.T, mean) * p['mean_scale']
    var = _hmm(batch_oh, out * out) * inv_cnt[:, None]
    scale = jax.lax.rsqrt(var + 1e-5) * p['weight']
    return out * _hmm(batch_oh.T, scale) + p['bias']


def _sort_aggr(x, batch, cnt, b, k):
    order = jnp.lexsort((-x[:, -1], batch))
    counts = cnt.astype(_i32)
    starts = jnp.concatenate(
        [jnp.zeros((1,), _i32), jnp.cumsum(counts)[:-1]])
    pos = starts[:, None] + jnp.arange(k, dtype=_i32)[None, :]
    valid = jnp.arange(k, dtype=_i32)[None, :] < counts[:, None]
    top_idx = order[jnp.clip(pos, 0, x.shape[0] - 1)]
    rows = x[top_idx.reshape(-1)].reshape(b, k, x.shape[1])
    rows = jnp.where(valid[:, :, None], rows, 0.0)
    return rows.reshape(b, k * x.shape[1])


_DSTF = [None]


def kernel(x, edge_index, batch, additional_feat, params):
    b = NUM_GRAPHS
    src, dst = edge_index[0], edge_index[1]
    pad = jnp.full((EPAD - N_EDGES,), N_NODES, _i32)
    src2 = jnp.concatenate([src, pad]).reshape(EPAD // K2, K2)
    dst_flat = jnp.concatenate([dst, pad])
    dst2 = dst_flat.reshape(EPAD // K2, K2)
    _DSTF[0] = dst_flat
    batch_oh = (batch[None, :] == jnp.arange(b, dtype=batch.dtype)[:, None]
                ).astype(_f32)
    cnt = batch_oh.sum(axis=1)
    inv_cnt = 1.0 / jnp.maximum(cnt, 1.0)

    ones_rows = jnp.ones((1, EPAD, DC), _f32)
    deg = _merge_halves(_scatter_call(ones_rows, *_half_idx(dst_flat)))[0, :, 0]
    dis = jnp.where(deg > 0, 1.0 / jnp.sqrt(jnp.maximum(deg, 1e-12)), 0.0)

    h = _gnorm(jax.nn.elu(_gat_layer(x, src2, dst2, params['gat1'], 64)),
               batch_oh, inv_cnt, params['gn1'], b)
    h = _gnorm(jax.nn.elu(_gat_layer(h, src2, dst2, params['gat2'], 128)),
               batch_oh, inv_cnt, params['gn2'], b)
    h = _gnorm(jax.nn.elu(_gat_layer(h, src2, dst2, params['gat3'], 512)),
               batch_oh, inv_cnt, params['gn3'], b)
    g = _gnorm(jax.nn.elu(_arma_layer(x, x, dis, src2, dst2, params['arma1'], 64)),
               batch_oh, inv_cnt, params['gn4'], b)
    g = _gnorm(jax.nn.elu(_arma_layer(g, g, dis, src2, dst2, params['arma2'], 128)),
               batch_oh, inv_cnt, params['gn5'], b)
    g = _gnorm(jax.nn.elu(_arma_layer(g, g, dis, src2, dst2, params['arma3'], 512)),
               batch_oh, inv_cnt, params['gn6'], b)
    gg = jnp.concatenate([h, g], axis=1)
    sums = _hmm(batch_oh, gg)
    pool = jnp.concatenate([_seg_max(gg, batch, b), sums * inv_cnt[:, None],
                            sums], axis=1)
    sa = _sort_aggr(gg, batch, cnt, b, 4)
    return _head(pool, sa, additional_feat.reshape(b, 9), params)


# speedup vs baseline: 1.0970x; 1.0925x over previous
import functools

import jax
import jax.numpy as jnp
from jax import lax
from jax.experimental import pallas as pl
from jax.experimental.pallas import tpu as pltpu
from jax.experimental.pallas import tpu_sc as plsc

N_NODES = 10000
N_EDGES = 160000
NUM_GRAPHS = 128

NTILES = 32
EPAD = 163840
EPT = EPAD // NTILES
K2 = 128
NPAD = 10016
NHALF = 5008
ACCR = 5024
NZB = 157

_f32 = jnp.float32
_i32 = jnp.int32

_MESH = plsc.VectorSubcoreMesh(core_axis_name="c", subcore_axis_name="s")


KQ = 6
KQS = 3


def _gather_body(nchunks, nidx, which, refs):
    tabs = refs[:nchunks]
    idx_hbm = refs[nchunks:nchunks + nidx]
    out = refs[nchunks + nidx]
    idx_v = refs[nchunks + nidx + 1:nchunks + nidx + 1 + nidx]
    o = nchunks + nidx + 1 + nidx
    rbufs = refs[o:o + 1]
    gsem, wsem = refs[o + 1], refs[o + 2]
    cid = lax.axis_index("c")
    sid = lax.axis_index("s")
    wid = cid * 16 + sid
    nb = EPT // K2

    rbuf = rbufs[0]
    for i in range(nidx):
        pltpu.sync_copy(idx_hbm[i].at[pl.ds(wid * nb, nb)], idx_v[i])
    for cc in range(nchunks):
        idx2 = idx_v[which[cc]]

        def sb(q, _):
            ds = [pltpu.async_copy(tabs[cc].at[idx2.at[q * KQ + j]],
                                   rbuf.at[pl.ds(j * K2, K2)], gsem)
                  for j in range(KQ)]
            for d in ds:
                d.wait()
            pltpu.async_copy(
                rbuf, out.at[cc, pl.ds(wid * EPT + q * (KQ * K2), KQ * K2)],
                wsem).wait()
            return 0

        lax.fori_loop(0, nb // KQ, sb, 0)


def _gather_call(tabs, idx2s, which, dc):
    nchunks = len(tabs)
    nidx = len(idx2s)

    def wrapped(*refs):
        _gather_body(nchunks, nidx, which, refs)

    return pl.kernel(
        wrapped,
        out_type=jax.ShapeDtypeStruct((nchunks, EPAD, dc), _f32),
        mesh=_MESH,
        scratch_types=[
            pltpu.VMEM((EPAD // K2 // NTILES, K2), _i32)
            for _ in range(nidx)
        ] + [pltpu.VMEM((KQ * K2, dc), _f32)] + [
            pltpu.SemaphoreType.DMA,
            pltpu.SemaphoreType.DMA,
        ],
    )(*tabs, *idx2s)


def _gs_body(nchunks, refs):
    tabs = refs[:nchunks]
    src2_hbm = refs[nchunks]
    idxl_hbm = refs[nchunks + 1]
    idxh_hbm = refs[nchunks + 2]
    out = refs[nchunks + 3]
    src2 = refs[nchunks + 4]
    idxa, idxb = refs[nchunks + 5], refs[nchunks + 6]
    rbufs = refs[nchunks + 7:nchunks + 7 + KQ]
    zblk = refs[nchunks + 7 + KQ]
    acc_s = refs[nchunks + 8 + KQ]
    gsem, ssem = refs[nchunks + 9 + KQ], refs[nchunks + 10 + KQ]
    idx_v = (idxa, idxb)
    cid = lax.axis_index("c")
    sid = lax.axis_index("s")
    wid = cid * 16 + sid
    nb = EPT // K2

    pltpu.sync_copy(src2_hbm.at[pl.ds(wid * nb, nb)], src2)
    pltpu.sync_copy(idxl_hbm.at[pl.ds(wid * nb, nb)], idxa)
    pltpu.sync_copy(idxh_hbm.at[pl.ds(wid * nb, nb)], idxb)

    def zb(i, _):
        for q in range(DC // 16):
            zblk[i, pl.ds(q * 16, 16)] = jnp.zeros((16,), _f32)
        return 0

    lax.fori_loop(0, NZB, zb, 0)

    for cc in range(nchunks):
        for hf in range(2):
            for r2 in range(2):
                pltpu.sync_copy(
                    zblk, acc_s.at[pl.ds((sid * 2 + r2) * NZB, NZB)])
            plsc.subcore_barrier()

            def sb(q, _):
                ds = [pltpu.async_copy(tabs[cc].at[src2.at[q * KQ + j]],
                                       rbufs[j], gsem) for j in range(KQ)]
                for d in ds:
                    d.wait()
                ws = [pltpu.async_copy(
                    rbufs[j], acc_s.at[idx_v[hf].at[q * KQ + j]],
                    ssem, add=True) for j in range(KQ)]
                for w in ws:
                    w.wait()
                return 0

            lax.fori_loop(0, nb // KQ, sb, 0)
            plsc.subcore_barrier()

            @pl.when(sid == 0)
            def _():
                pltpu.sync_copy(acc_s, out.at[cc, cid, hf])

            plsc.subcore_barrier()


def _gs_call(tabs, src2, idx_lo, idx_hi):
    nchunks = len(tabs)

    def wrapped(*refs):
        _gs_body(nchunks, refs)

    return pl.kernel(
        wrapped,
        out_type=jax.ShapeDtypeStruct((nchunks, 2, 2, ACCR, DC), _f32),
        mesh=_MESH,
        scratch_types=[
            pltpu.VMEM((EPAD // K2 // NTILES, K2), _i32),
            pltpu.VMEM((EPAD // K2 // NTILES, K2), _i32),
            pltpu.VMEM((EPAD // K2 // NTILES, K2), _i32),
        ] + [pltpu.VMEM((K2, DC), _f32) for _ in range(KQ)] + [
            pltpu.VMEM((NZB, DC), _f32),
            pltpu.VMEM_SHARED((ACCR, DC), _f32),
            pltpu.SemaphoreType.DMA,
            pltpu.SemaphoreType.DMA,
        ],
    )(*tabs, src2, idx_lo, idx_hi)


def _scatter_body(nchunks, refs):
    wg_hbm = refs[0]
    idx_hbm = (refs[1], refs[2])
    out = refs[3]
    idxa, idxb = refs[4], refs[5]
    rbuf = refs[6]
    zblk, acc_s = refs[7], refs[8]
    gsem, ssem = refs[9], refs[10]
    idx_v = (idxa, idxb)
    cid = lax.axis_index("c")
    sid = lax.axis_index("s")
    wid = cid * 16 + sid
    nb = EPT // K2

    for hf in range(2):
        pltpu.sync_copy(idx_hbm[hf].at[pl.ds(wid * nb, nb)], idx_v[hf])

    def zb(i, _):
        for q in range(DC // 16):
            zblk[i, pl.ds(q * 16, 16)] = jnp.zeros((16,), _f32)
        return 0

    lax.fori_loop(0, NZB, zb, 0)

    for cc in range(nchunks):
        for hf in range(2):
            for r2 in range(2):
                pltpu.sync_copy(
                    zblk, acc_s.at[pl.ds((sid * 2 + r2) * NZB, NZB)])
            plsc.subcore_barrier()

            def sb(q, _):
                pltpu.async_copy(
                    wg_hbm.at[cc, pl.ds(wid * EPT + q * (KQS * K2), KQS * K2)],
                    rbuf, gsem).wait()
                ws = [pltpu.async_copy(
                    rbuf.at[pl.ds(j * K2, K2)],
                    acc_s.at[idx_v[hf].at[q * KQS + j]],
                    ssem, add=True) for j in range(KQS)]
                for w in ws:
                    w.wait()
                return 0

            lax.fori_loop(0, nb // KQS, sb, 0)
            plsc.subcore_barrier()

            @pl.when(sid == 0)
            def _():
                pltpu.sync_copy(acc_s, out.at[cc, cid, hf])

            plsc.subcore_barrier()


def _scatter_call(wg, idx_lo, idx_hi):
    nchunks = wg.shape[0]

    def wrapped(*refs):
        _scatter_body(nchunks, refs)

    return pl.kernel(
        wrapped,
        out_type=jax.ShapeDtypeStruct((nchunks, 2, 2, ACCR, DC), _f32),
        mesh=_MESH,
        scratch_types=[
            pltpu.VMEM((EPAD // K2 // NTILES, K2), _i32),
            pltpu.VMEM((EPAD // K2 // NTILES, K2), _i32),
            pltpu.VMEM((KQS * K2, DC), _f32),
            pltpu.VMEM((NZB, DC), _f32),
            pltpu.VMEM_SHARED((ACCR, DC), _f32),
            pltpu.SemaphoreType.DMA,
            pltpu.SemaphoreType.DMA,
        ],
    )(wg, idx_lo, idx_hi)


def _half_idx(dst_flat):
    lo = jnp.where(dst_flat < NHALF, dst_flat, NHALF)
    inhi = (dst_flat >= NHALF) & (dst_flat < N_NODES)
    hi = jnp.where(inhi, dst_flat - NHALF, NHALF)
    shp = (EPAD // K2, K2)
    return lo.reshape(shp), hi.reshape(shp)


def _merge_halves(acc):
    acc = acc.sum(axis=1)
    return jnp.concatenate([acc[:, 0, :NHALF], acc[:, 1, :N_NODES - NHALF]], axis=1)


DC = 128


def _chunk(a):
    do = a.shape[1]
    return [a[:, i * DC:(i + 1) * DC] for i in range(do // DC)]


def _aug(a):
    dop = max(DC, a.shape[1])
    return jnp.concatenate([
        jnp.pad(a, ((0, 0), (0, dop - a.shape[1]))),
        jnp.zeros((NPAD - N_NODES, dop), _f32)])


def _merge_num(num_p):
    num = num_p.sum(axis=1)[:, :N_NODES]
    return jnp.moveaxis(num, 0, 1).reshape(N_NODES, -1)


def _leaky(z):
    return 0.6 * z + 0.4 * jnp.abs(z)


def _gat_layer(h, src2, dst2, p, do):
    dop = max(DC, do)
    attp = jnp.pad(p['att'], (0, dop - do))
    xl = h @ p['Wl'] + p['bl']
    xr = h @ p['Wr'] + p['br']
    xla = _aug(xl)
    xra = _aug(xr)
    cl = _chunk(xla)
    cr = _chunk(xra)
    both = _gather_call(cl + cr, [src2, dst2],
                        [0] * len(cl) + [1] * len(cr), DC)
    xlg, xrg = both[:len(cl)], both[len(cl):]
    xlg_f = jnp.moveaxis(xlg, 0, 1).reshape(EPAD, dop)
    zg = xlg_f + jnp.moveaxis(xrg, 0, 1).reshape(EPAD, dop)
    ex = jnp.exp(_leaky(zg) @ attp)
    wg = ex[:, None] * xlg_f
    wg_c = jnp.moveaxis(wg.reshape(EPAD, dop // DC, DC), 1, 0)
    den_c = jnp.broadcast_to(ex[:, None], (EPAD, DC))[None]
    acc = _scatter_call(jnp.concatenate([wg_c, den_c]), *_half_idx(_DSTF[0]))
    mh = _merge_halves(acc)
    num = jnp.moveaxis(mh[:-1], 0, 1).reshape(N_NODES, -1)[:, :do]
    den = mh[-1][:, 0]
    exs = jnp.exp(_leaky(xl + xr) @ p['att'])
    out = (num + exs[:, None] * xl) / (den + exs + 1e-16)[:, None]
    return out + p['bias']


def _arma_layer(g, x_skip, dis, src2, dst2, p, do):
    hp = _aug(dis[:, None] * (g @ p['W']))
    hg = _gather_call(_chunk(hp), [src2], [0] * len(_chunk(hp)), DC)
    acc = _scatter_call(hg, *_half_idx(_DSTF[0]))
    mh = _merge_halves(acc)
    agg = dis[:, None] * jnp.moveaxis(mh, 0, 1).reshape(N_NODES, -1)[:, :do]
    return jax.nn.relu(agg + x_skip @ p['V'] + p['bias'])


def _head_body(pool_ref, sa_ref, af_ref, w1, b1, w2, b2, w4, b4, w3, out_ref):
    z1 = jnp.dot(pool_ref[...], w1[...], preferred_element_type=_f32) + b1[...]
    z2 = jnp.dot(sa_ref[...], w2[...], preferred_element_type=_f32) + b2[...]
    z3 = jnp.dot(af_ref[...], w4[...], preferred_element_type=_f32) + b4[...]
    w3a = w3[...][:512]
    w3b = w3[...][512:1024]
    w3c = w3[...][1024:]
    out_ref[...] = (
        jnp.dot(z1, w3a, preferred_element_type=_f32)
        + jnp.dot(z2, w3b, preferred_element_type=_f32)
        + jnp.dot(z3, w3c, preferred_element_type=_f32)
    )


def _head(pool, sa, af, p):
    return pl.pallas_call(
        _head_body,
        out_shape=jax.ShapeDtypeStruct((NUM_GRAPHS, 1), _f32),
    )(pool, sa, af,
      p['lin1']['W'], p['lin1']['b'],
      p['lin2']['W'], p['lin2']['b'],
      p['lin4']['W'], p['lin4']['b'],
      p['lin3']['W'])


def _seg_sum(d, s, n):
    return jax.ops.segment_sum(d, s, num_segments=n)


def _seg_mean(d, s, n):
    tot = jax.ops.segment_sum(d, s, num_segments=n)
    cnt = jax.ops.segment_sum(jnp.ones(d.shape[:1], d.dtype), s, num_segments=n)
    cnt = jnp.maximum(cnt, 1.0)
    return tot / cnt.reshape((-1,) + (1,) * (d.ndim - 1))


def _seg_max(d, s, n):
    m = jax.ops.segment_max(d, s, num_segments=n)
    return jnp.where(jnp.isfinite(m), m, 0.0)


def _hmm(a, b):
    return jnp.matmul(a, b, precision=jax.lax.Precision.HIGHEST)


def _gnorm(x, batch_oh, inv_cnt, p, b):
    mean = _hmm(batch_oh, x) * inv_cnt[:, None]
    out = x - _hmm(batch_oh.T, mean) * p['mean_scale']
    var = _hmm(batch_oh, out * out) * inv_cnt[:, None]
    scale = jax.lax.rsqrt(var + 1e-5) * p['weight']
    return out * _hmm(batch_oh.T, scale) + p['bias']


def _sort_aggr(x, batch, cnt, b, k):
    order = jnp.lexsort((-x[:, -1], batch))
    counts = cnt.astype(_i32)
    starts = jnp.concatenate(
        [jnp.zeros((1,), _i32), jnp.cumsum(counts)[:-1]])
    pos = starts[:, None] + jnp.arange(k, dtype=_i32)[None, :]
    valid = jnp.arange(k, dtype=_i32)[None, :] < counts[:, None]
    top_idx = order[jnp.clip(pos, 0, x.shape[0] - 1)]
    rows = x[top_idx.reshape(-1)].reshape(b, k, x.shape[1])
    rows = jnp.where(valid[:, :, None], rows, 0.0)
    return rows.reshape(b, k * x.shape[1])


_DSTF = [None]


def kernel(x, edge_index, batch, additional_feat, params):
    b = NUM_GRAPHS
    src, dst = edge_index[0], edge_index[1]
    pad = jnp.full((EPAD - N_EDGES,), N_NODES, _i32)
    src2 = jnp.concatenate([src, pad]).reshape(EPAD // K2, K2)
    dst_flat = jnp.concatenate([dst, pad])
    dst2 = dst_flat.reshape(EPAD // K2, K2)
    _DSTF[0] = dst_flat
    batch_oh = (batch[None, :] == jnp.arange(b, dtype=batch.dtype)[:, None]
                ).astype(_f32)
    cnt = batch_oh.sum(axis=1)
    inv_cnt = 1.0 / jnp.maximum(cnt, 1.0)

    ones_rows = jnp.ones((1, EPAD, DC), _f32)
    deg = _merge_halves(_scatter_call(ones_rows, *_half_idx(dst_flat)))[0, :, 0]
    dis = jnp.where(deg > 0, 1.0 / jnp.sqrt(jnp.maximum(deg, 1e-12)), 0.0)

    h = _gnorm(jax.nn.elu(_gat_layer(x, src2, dst2, params['gat1'], 64)),
               batch_oh, inv_cnt, params['gn1'], b)
    h = _gnorm(jax.nn.elu(_gat_layer(h, src2, dst2, params['gat2'], 128)),
               batch_oh, inv_cnt, params['gn2'], b)
    h = _gnorm(jax.nn.elu(_gat_layer(h, src2, dst2, params['gat3'], 512)),
               batch_oh, inv_cnt, params['gn3'], b)
    g = _gnorm(jax.nn.elu(_arma_layer(x, x, dis, src2, dst2, params['arma1'], 64)),
               batch_oh, inv_cnt, params['gn4'], b)
    g = _gnorm(jax.nn.elu(_arma_layer(g, g, dis, src2, dst2, params['arma2'], 128)),
               batch_oh, inv_cnt, params['gn5'], b)
    g = _gnorm(jax.nn.elu(_arma_layer(g, g, dis, src2, dst2, params['arma3'], 512)),
               batch_oh, inv_cnt, params['gn6'], b)
    gg = jnp.concatenate([h, g], axis=1)
    sums = _hmm(batch_oh, gg)
    pool = jnp.concatenate([_seg_max(gg, batch, b), sums * inv_cnt[:, None],
                            sums], axis=1)
    sa = _sort_aggr(gg, batch, cnt, b, 4)
    return _head(pool, sa, additional_feat.reshape(b, 9), params)
